# baseline (speedup 1.0000x reference)
import functools

import jax
import jax.numpy as jnp
from jax import lax
from jax.experimental import pallas as pl
from jax.experimental.pallas import tpu as pltpu
from jax.experimental.pallas import tpu_sc as plsc

B = 32
N = 2048
S = 20
KP1 = 11
L = 16
CHUNKS = N // L
A = 4

_mesh = plsc.VectorSubcoreMesh(core_axis_name="c", subcore_axis_name="s")


def _bmerge(a, b):
    return lax.sort(jnp.minimum(a, lax.rev(b, (0,))))


@functools.partial(
    pl.kernel,
    out_type=jax.ShapeDtypeStruct((B, S * L), jnp.float32),
    mesh=_mesh,
    compiler_params=pltpu.CompilerParams(needs_layout_passes=False),
    scratch_types=[
        pltpu.VMEM((N,), jnp.float32),
        pltpu.VMEM((N,), jnp.float32),
        pltpu.VMEM((N,), jnp.float32),
        pltpu.VMEM((N,), jnp.float32),
        pltpu.VMEM((S * L,), jnp.float32),
    ],
)
def _sc_fps_topk(xyz_hbm, out_hbm, xv, yv, zv, dv, bv):
    b = lax.axis_index("s") * 2 + lax.axis_index("c")
    pltpu.sync_copy(xyz_hbm.at[0, b], xv)
    pltpu.sync_copy(xyz_hbm.at[1, b], yv)
    pltpu.sync_copy(xyz_hbm.at[2, b], zv)

    iota = lax.iota(jnp.int32, L)
    inf16 = jnp.full((L,), jnp.inf, jnp.float32)

    def init_dist(j, carry):
        dv[pl.ds(j * L, L)] = jnp.full((L,), 1e10, jnp.float32)
        return carry

    lax.fori_loop(0, CHUNKS, init_dist, 0)

    def fps_step(i, carry):
        cx, cy, cz = carry

        def group(g, ch):
            accs = list(ch[:A])
            mx, mi = ch[A], ch[A + 1]
            for u in range(A):
                base = (g * A + u) * L
                dx = xv[pl.ds(base, L)] - cx
                dy = yv[pl.ds(base, L)] - cy
                dz = zv[pl.ds(base, L)] - cz
                d2 = dx * dx + dy * dy + dz * dz
                nd = jnp.minimum(dv[pl.ds(base, L)], d2)
                dv[pl.ds(base, L)] = nd
                pos = iota + base
                upd = nd > mx
                mx = jnp.where(upd, nd, mx)
                mi = jnp.where(upd, pos, mi)
                accs[u] = _bmerge(accs[u], lax.sort(d2))
            return (*accs, mx, mi)

        init = (*([inf16] * A),
                jnp.full((L,), -1.0, jnp.float32),
                jnp.zeros((L,), jnp.int32))
        res = lax.fori_loop(0, CHUNKS // A, group, init)
        accs, mx, mi = list(res[:A]), res[A], res[A + 1]

        gmax = jnp.max(mx)
        gidx = jnp.min(jnp.where(mx == gmax, mi, N))

        bv[pl.ds(i * L, L)] = _bmerge(_bmerge(accs[0], accs[1]),
                                      _bmerge(accs[2], accs[3]))

        gv = jnp.full((L,), gidx, jnp.int32)
        return (plsc.load_gather(xv, [gv]),
                plsc.load_gather(yv, [gv]),
                plsc.load_gather(zv, [gv]))

    c0 = (jnp.full((L,), xv[pl.ds(0, L)][0]),
          jnp.full((L,), yv[pl.ds(0, L)][0]),
          jnp.full((L,), zv[pl.ds(0, L)][0]))
    lax.fori_loop(0, S, fps_step, c0)
    pltpu.sync_copy(bv, out_hbm.at[b])


def _stats_body(a_ref, o_ref):
    x = a_ref[...]
    d = jnp.sqrt(jnp.maximum(x, 1e-12))
    lane = lax.broadcasted_iota(jnp.int32, (B * S, L), 1)
    dm = jnp.where(lane < KP1, d, 0.0)
    rowsum = jnp.sum(dm, axis=1, keepdims=True)
    rowmin = jnp.sum(jnp.where(lane == 0, d, 0.0), axis=1, keepdims=True)
    overall_mean = (jnp.sum(rowsum) - jnp.sum(rowmin)) / (B * S * (KP1 - 1))
    rm = rowsum / KP1
    mean_rm = jnp.sum(rm) / (B * S)
    var = jnp.sum((rm - mean_rm) ** 2) / (B * S - 1)
    o_ref[...] = jnp.reshape(var / (overall_mean * overall_mean), (1, 1))


def kernel(pcs):
    xyz = jnp.transpose(pcs, (2, 0, 1))
    best = _sc_fps_topk(xyz)
    out = pl.pallas_call(
        _stats_body,
        out_shape=jax.ShapeDtypeStruct((1, 1), jnp.float32),
    )(best.reshape(B * S, L))
    return out[0, 0]


# speedup vs baseline: 1.4893x; 1.4893x over previous
import functools

import jax
import jax.numpy as jnp
from jax import lax
from jax.experimental import pallas as pl
from jax.experimental.pallas import tpu as pltpu
from jax.experimental.pallas import tpu_sc as plsc

B = 32
N = 2048
S = 20
KP1 = 11
L = 16
CHUNKS = N // L
A = 4

_mesh = plsc.VectorSubcoreMesh(core_axis_name="c", subcore_axis_name="s")


def _bmerge(a, b):
    return lax.sort(jnp.minimum(a, lax.rev(b, (0,))))


@functools.partial(
    pl.kernel,
    out_type=jax.ShapeDtypeStruct((B, S * L), jnp.float32),
    mesh=_mesh,
    compiler_params=pltpu.CompilerParams(needs_layout_passes=False),
    scratch_types=[
        pltpu.VMEM((N,), jnp.float32),
        pltpu.VMEM((N,), jnp.float32),
        pltpu.VMEM((N,), jnp.float32),
        pltpu.VMEM((N,), jnp.float32),
        pltpu.VMEM((S * L,), jnp.float32),
    ],
)
def _sc_fps_topk(xyz_hbm, out_hbm, xv, yv, zv, dv, bv):
    b = lax.axis_index("s") * 2 + lax.axis_index("c")
    pltpu.sync_copy(xyz_hbm.at[0, b], xv)
    pltpu.sync_copy(xyz_hbm.at[1, b], yv)
    pltpu.sync_copy(xyz_hbm.at[2, b], zv)

    iota = lax.iota(jnp.int32, L)
    inf16 = jnp.full((L,), jnp.inf, jnp.float32)

    @plsc.parallel_loop(0, N, step=4 * L, unroll=2)
    def _init(off):
        for u in range(4):
            dv[pl.ds(off + u * L, L)] = jnp.full((L,), 1e10, jnp.float32)

    def fps_step(i, carry):
        cx, cy, cz = carry

        init = (*([inf16] * A),
                jnp.full((L,), -1.0, jnp.float32),
                jnp.zeros((L,), jnp.int32))

        @plsc.parallel_loop(0, N, step=A * L, unroll=2, carry=init)
        def res(off, ch):
            accs = list(ch[:A])
            mx, mi = ch[A], ch[A + 1]
            for u in range(A):
                base = off + u * L
                dx = xv[pl.ds(base, L)] - cx
                dy = yv[pl.ds(base, L)] - cy
                dz = zv[pl.ds(base, L)] - cz
                d2 = dx * dx + dy * dy + dz * dz
                nd = jnp.minimum(dv[pl.ds(base, L)], d2)
                dv[pl.ds(base, L)] = nd
                pos = iota + base
                upd = nd > mx
                mx = jnp.where(upd, nd, mx)
                mi = jnp.where(upd, pos, mi)
                accs[u] = _bmerge(accs[u], lax.sort(d2))
            return (*accs, mx, mi)

        accs, mx, mi = list(res[:A]), res[A], res[A + 1]

        gmax = jnp.max(mx)
        gidx = jnp.min(jnp.where(mx == gmax, mi, N))

        bv[pl.ds(i * L, L)] = _bmerge(_bmerge(accs[0], accs[1]),
                                      _bmerge(accs[2], accs[3]))

        gv = jnp.full((L,), gidx, jnp.int32)
        return (plsc.load_gather(xv, [gv]),
                plsc.load_gather(yv, [gv]),
                plsc.load_gather(zv, [gv]))

    c0 = (jnp.full((L,), xv[pl.ds(0, L)][0]),
          jnp.full((L,), yv[pl.ds(0, L)][0]),
          jnp.full((L,), zv[pl.ds(0, L)][0]))
    lax.fori_loop(0, S, fps_step, c0)
    pltpu.sync_copy(bv, out_hbm.at[b])


def _stats_body(a_ref, o_ref):
    x = a_ref[...]
    d = jnp.sqrt(jnp.maximum(x, 1e-12))
    lane = lax.broadcasted_iota(jnp.int32, (B * S, L), 1)
    dm = jnp.where(lane < KP1, d, 0.0)
    rowsum = jnp.sum(dm, axis=1, keepdims=True)
    rowmin = jnp.sum(jnp.where(lane == 0, d, 0.0), axis=1, keepdims=True)
    overall_mean = (jnp.sum(rowsum) - jnp.sum(rowmin)) / (B * S * (KP1 - 1))
    rm = rowsum / KP1
    mean_rm = jnp.sum(rm) / (B * S)
    var = jnp.sum((rm - mean_rm) ** 2) / (B * S - 1)
    o_ref[...] = jnp.reshape(var / (overall_mean * overall_mean), (1, 1))


def kernel(pcs):
    xyz = jnp.transpose(pcs, (2, 0, 1))
    best = _sc_fps_topk(xyz)
    out = pl.pallas_call(
        _stats_body,
        out_shape=jax.ShapeDtypeStruct((1, 1), jnp.float32),
    )(best.reshape(B * S, L))
    return out[0, 0]
